# baseline (speedup 1.0000x reference)
import functools

import numpy as np
import jax
import jax.numpy as jnp
from jax import lax
from jax.experimental import pallas as pl
from jax.experimental.pallas import tpu as pltpu
from jax.experimental.pallas import tpu_sc as plsc

B = 8
NB = B // 2
N = 512
MSG = 512
TOPK = 16
EPS_BN = 1e-5
INV_SQRT2 = float(1.0 / np.sqrt(2.0))
BN_SCALE = float(1.0 / np.sqrt(1.0 + EPS_BN))

NC = 2
NS = 16
LANES = 16
NW = NC * NS
TOTALH = NB * N
NPW = TOTALH // NW
VCHUNKS = MSG // LANES
NBUF = 4
XCHUNK = 32
GPC = XCHUNK // NBUF


def _topk_body(a_ref, an_ref, idx_ref, mn_ref):
    a = a_ref[0]
    colid = lax.broadcasted_iota(jnp.int32, (N, N), 1)
    work = a
    vals = []
    idxs = []
    for _ in range(TOPK):
        m = jnp.max(work, axis=1, keepdims=True)
        eq = work == m
        idxk = jnp.min(jnp.where(eq, colid, N), axis=1, keepdims=True)
        vals.append(m)
        idxs.append(idxk)
        work = jnp.where(colid == idxk, -jnp.inf, work)
    v = jnp.concatenate(vals, axis=1)
    ix = jnp.concatenate(idxs, axis=1)
    inv = 1.0 / (jnp.sum(v, axis=1, keepdims=True) + 1e-12)
    an_ref[0] = v * inv
    idx_ref[0] = ix + pl.program_id(0) * N
    mn_ref[0] = jnp.where(work != a, a * inv, 0.0)


def _topk(A):
    return pl.pallas_call(
        _topk_body,
        grid=(NB,),
        in_specs=[pl.BlockSpec((1, N, N), lambda i: (i, 0, 0))],
        out_specs=[pl.BlockSpec((1, N, TOPK), lambda i: (i, 0, 0)),
                   pl.BlockSpec((1, N, TOPK), lambda i: (i, 0, 0)),
                   pl.BlockSpec((1, N, N), lambda i: (i, 0, 0))],
        out_shape=[jax.ShapeDtypeStruct((NB, N, TOPK), jnp.float32),
                   jax.ShapeDtypeStruct((NB, N, TOPK), jnp.int32),
                   jax.ShapeDtypeStruct((NB, N, N), jnp.float32)],
    )(A)


def _proj_body(x_ref, w_ref, b_ref, mn_ref, an_ref, cc_ref, pre_ref, lin_ref, xb_ref):
    x = x_ref[0]
    xl = lax.dot_general(
        x, w_ref[...], (((1,), (1,)), ((), ())),
        preferred_element_type=jnp.float32) + b_ref[...][None, :]
    gg = lax.dot_general(
        mn_ref[0], xl, (((1,), (0,)), ((), ())),
        preferred_element_type=jnp.float32)
    sume = jnp.sum(an_ref[0], axis=1, keepdims=True)
    c1 = cc_ref[0, :][None, :]
    c2 = cc_ref[1, :][None, :]
    xlc1 = xl * c1
    pre_ref[0] = xlc1 + c2
    xb_ref[0] = xlc1
    lin_ref[0] = 0.505 * ((sume * xl + gg) * c1 + sume * c2)


def _proj(X, W, b, Mn, anb, cc, din):
    return pl.pallas_call(
        _proj_body,
        grid=(NB,),
        in_specs=[pl.BlockSpec((1, N, din), lambda i: (i, 0, 0)),
                  pl.BlockSpec((MSG, din), lambda i: (0, 0)),
                  pl.BlockSpec((MSG,), lambda i: (0,)),
                  pl.BlockSpec((1, N, N), lambda i: (i, 0, 0)),
                  pl.BlockSpec((1, N, TOPK), lambda i: (i, 0, 0)),
                  pl.BlockSpec((2, MSG), lambda i: (0, 0))],
        out_specs=[pl.BlockSpec((1, N, MSG), lambda i: (i, 0, 0)),
                   pl.BlockSpec((1, N, MSG), lambda i: (i, 0, 0)),
                   pl.BlockSpec((1, N, MSG), lambda i: (i, 0, 0))],
        out_shape=[jax.ShapeDtypeStruct((NB, N, MSG), jnp.float32),
                   jax.ShapeDtypeStruct((NB, N, MSG), jnp.float32),
                   jax.ShapeDtypeStruct((NB, N, MSG), jnp.float32)],
    )(X, W, b, Mn, anb, cc)


def _update_body(x_ref, m_ref, l_ref, w_ref, g_ref, be_ref, o_ref, *, din):
    x = x_ref[0]
    m = l_ref[0] + 0.495 * m_ref[0]
    w = w_ref[...]
    u = lax.dot_general(x, w[:, :din], (((1,), (1,)), ((), ())),
                        preferred_element_type=jnp.float32)
    u = u + lax.dot_general(m, w[:, din:], (((1,), (1,)), ((), ())),
                            preferred_element_type=jnp.float32)
    u = u * (g_ref[...] * BN_SCALE)[None, :] + be_ref[...][None, :]
    u = jnp.where(u >= 0, u, 0.01 * u)
    nrm = jnp.maximum(jnp.sqrt(jnp.sum(u * u, axis=1, keepdims=True)), 1e-12)
    o_ref[0, :, :din] = x * INV_SQRT2
    o_ref[0, :, din:] = u * (INV_SQRT2 / nrm)


def _update(X, msg, lin, W, g, be, din):
    return pl.pallas_call(
        functools.partial(_update_body, din=din),
        grid=(NB,),
        in_specs=[pl.BlockSpec((1, N, din), lambda i: (i, 0, 0)),
                  pl.BlockSpec((1, N, MSG), lambda i: (i, 0, 0)),
                  pl.BlockSpec((1, N, MSG), lambda i: (i, 0, 0)),
                  pl.BlockSpec((MSG, din + MSG), lambda i: (0, 0)),
                  pl.BlockSpec((MSG,), lambda i: (0,)),
                  pl.BlockSpec((MSG,), lambda i: (0,))],
        out_specs=pl.BlockSpec((1, N, din + MSG), lambda i: (i, 0, 0)),
        out_shape=jax.ShapeDtypeStruct((NB, N, din + MSG), jnp.float32),
    )(X, msg, lin, W, g, be)


def _final_body(x_ref, m_ref, l_ref, w_ref, b_ref, o_ref, *, din):
    x = x_ref[0]
    m = l_ref[0] + 0.495 * m_ref[0]
    w = w_ref[...]
    u = lax.dot_general(x, w[:, :din], (((1,), (1,)), ((), ())),
                        preferred_element_type=jnp.float32)
    u = u + lax.dot_general(m, w[:, din:], (((1,), (1,)), ((), ())),
                            preferred_element_type=jnp.float32)
    o_ref[0] = u + b_ref[...][None, :]


def _final(X, msg, lin, W, b, din):
    return pl.pallas_call(
        functools.partial(_final_body, din=din),
        grid=(NB,),
        in_specs=[pl.BlockSpec((1, N, din), lambda i: (i, 0, 0)),
                  pl.BlockSpec((1, N, MSG), lambda i: (i, 0, 0)),
                  pl.BlockSpec((1, N, MSG), lambda i: (i, 0, 0)),
                  pl.BlockSpec((MSG, din + MSG), lambda i: (0, 0)),
                  pl.BlockSpec((MSG,), lambda i: (0,))],
        out_specs=pl.BlockSpec((1, N, MSG), lambda i: (i, 0, 0)),
        out_shape=jax.ShapeDtypeStruct((NB, N, MSG), jnp.float32),
    )(X, msg, lin, W, b)


def _sc_msg_body(xl_hbm, xb_hbm, idx_hbm, an_hbm, out_hbm,
                 idx_v, an_v, xi_v, rows0, rows1, rows2, rows3,
                 msg0, msg1, msg2, msg3,
                 semg0, semg1, semg2, semg3, semo0, semo1, semo2, semo3):
    c = lax.axis_index("c")
    s = lax.axis_index("s")
    wid = s * NC + c
    base = wid * NPW

    pltpu.sync_copy(idx_hbm.at[pl.ds(base, NPW)], idx_v)
    pltpu.sync_copy(an_hbm.at[pl.ds(base, NPW)], an_v)

    rows = [rows0, rows1, rows2, rows3]
    msgs = [msg0, msg1, msg2, msg3]
    semg = [semg0, semg1, semg2, semg3]
    semo = [semo0, semo1, semo2, semo3]
    for p in range(NBUF):
        pltpu.make_async_copy(xl_hbm.at[idx_v.at[p]], rows[p], semg[p]).start()

    def compute_node(n, rowsb, msgb):
        av = an_v[n, :]
        aks = [jnp.broadcast_to(av[k], (LANES,)) for k in range(TOPK)]
        nl = lax.rem(n, XCHUNK)

        def vloop(v, carry):
            sl = pl.ds(v * LANES, LANES)
            basev = xi_v[nl, sl]
            acc = jnp.zeros((LANES,), jnp.float32)
            for k in range(TOPK):
                h = jnp.abs(rowsb[k, sl] + basev)
                acc = acc + aks[k] * h
            msgb[0, sl] = acc
            return carry
        lax.fori_loop(0, VCHUNKS, vloop, 0, unroll=4)

    def do_node(n, rowsb, msgb, sg, so):
        pltpu.make_async_copy(xl_hbm.at[idx_v.at[n]], rowsb, sg).wait()

        @pl.when(n >= NBUF)
        def _drain():
            pltpu.make_async_copy(
                msgb, out_hbm.at[pl.ds(base + n - NBUF, 1)], so).wait()

        compute_node(n, rowsb, msgb)

        @pl.when(n + NBUF < NPW)
        def _prefetch():
            pltpu.make_async_copy(xl_hbm.at[idx_v.at[n + NBUF]], rowsb, sg).start()

        pltpu.make_async_copy(msgb, out_hbm.at[pl.ds(base + n, 1)], so).start()

    def group(t, carry):
        @pl.when(lax.rem(t, GPC) == 0)
        def _xi_refresh():
            cstart = lax.div(t, GPC) * XCHUNK
            pltpu.sync_copy(xb_hbm.at[pl.ds(base + cstart, XCHUNK)], xi_v)

        n0 = NBUF * t
        for p in range(NBUF):
            do_node(n0 + p, rows[p], msgs[p], semg[p], semo[p])
        return carry

    lax.fori_loop(0, NPW // NBUF, group, 0)
    for p in range(NBUF):
        pltpu.make_async_copy(
            msgs[p], out_hbm.at[pl.ds(base + NPW - NBUF + p, 1)], semo[p]).wait()


def _sc_msg(xl_flat, xb_flat, idx_flat, an_flat):
    return pl.kernel(
        _sc_msg_body,
        out_type=jax.ShapeDtypeStruct((TOTALH, MSG), jnp.float32),
        mesh=plsc.VectorSubcoreMesh(core_axis_name="c", subcore_axis_name="s",
                                    num_cores=NC, num_subcores=NS),
        scratch_types=[
            pltpu.VMEM((NPW, TOPK), jnp.int32),
            pltpu.VMEM((NPW, TOPK), jnp.float32),
            pltpu.VMEM((XCHUNK, MSG), jnp.float32),
            pltpu.VMEM((TOPK, MSG), jnp.float32),
            pltpu.VMEM((TOPK, MSG), jnp.float32),
            pltpu.VMEM((TOPK, MSG), jnp.float32),
            pltpu.VMEM((TOPK, MSG), jnp.float32),
            pltpu.VMEM((1, MSG), jnp.float32),
            pltpu.VMEM((1, MSG), jnp.float32),
            pltpu.VMEM((1, MSG), jnp.float32),
            pltpu.VMEM((1, MSG), jnp.float32),
            pltpu.SemaphoreType.DMA,
            pltpu.SemaphoreType.DMA,
            pltpu.SemaphoreType.DMA,
            pltpu.SemaphoreType.DMA,
            pltpu.SemaphoreType.DMA,
            pltpu.SemaphoreType.DMA,
            pltpu.SemaphoreType.DMA,
            pltpu.SemaphoreType.DMA,
        ],
    )(xl_flat, xb_flat, idx_flat, an_flat)


def kernel(X_input, adjacency_matrix,
           W_agg0, b_agg0, g_agg0, be_agg0,
           W_agg1, b_agg1, g_agg1, be_agg1,
           W_agg2, b_agg2, g_agg2, be_agg2,
           W_upd0, g_upd0, be_upd0,
           W_upd1, g_upd1, be_upd1,
           W_fin, b_fin):
    A = adjacency_matrix.reshape(B, N, N)
    tk = [_topk(A[:NB]), _topk(A[NB:])]
    an_hh = [tk[0][0], tk[1][0]]
    Mn_h = [tk[0][2], tk[1][2]]
    an_h = [an_hh[0].reshape(TOTALH, TOPK), an_hh[1].reshape(TOTALH, TOPK)]
    idx_h = [tk[0][1].reshape(TOTALH, TOPK), tk[1][1].reshape(TOTALH, TOPK)]

    aggs = [(W_agg0, b_agg0, g_agg0, be_agg0),
            (W_agg1, b_agg1, g_agg1, be_agg1),
            (W_agg2, b_agg2, g_agg2, be_agg2)]
    upds = [(W_upd0, g_upd0, be_upd0), (W_upd1, g_upd1, be_upd1)]

    Xh = [X_input[:NB], X_input[NB:]]
    din = MSG
    outs = None
    for layer in range(3):
        W, b, g, be = aggs[layer]
        cc = jnp.stack([g * BN_SCALE, be])
        pl_ = [_proj(Xh[h], W, b, Mn_h[h], an_hh[h], cc, din)
               for h in range(2)]
        msg = [_sc_msg(pl_[h][0].reshape(TOTALH, MSG),
                       pl_[h][2].reshape(TOTALH, MSG),
                       idx_h[h], an_h[h]).reshape(NB, N, MSG)
               for h in range(2)]
        if layer < 2:
            Wu, gu, beu = upds[layer]
            Xh = [_update(Xh[h], msg[h], pl_[h][1], Wu, gu, beu, din)
                  for h in range(2)]
            din += MSG
        else:
            outs = [_final(Xh[h], msg[h], pl_[h][1], W_fin, b_fin, din)
                    for h in range(2)]
    return jnp.concatenate(outs, axis=0)


# speedup vs baseline: 1.1493x; 1.1493x over previous
import functools

import numpy as np
import jax
import jax.numpy as jnp
from jax import lax
from jax.experimental import pallas as pl
from jax.experimental.pallas import tpu as pltpu
from jax.experimental.pallas import tpu_sc as plsc

B = 8
NB = B // 2
N = 512
MSG = 512
TOPK = 16
EPS_BN = 1e-5
INV_SQRT2 = float(1.0 / np.sqrt(2.0))
BN_SCALE = float(1.0 / np.sqrt(1.0 + EPS_BN))

NC = 2
NS = 16
LANES = 16
NW = NC * NS
TOTALH = NB * N
NPW = TOTALH // NW
VCHUNKS = MSG // LANES
NBUF = 8
XCHUNK = 32
GPC = XCHUNK // NBUF


def _topk_body(a_ref, an_ref, idx_ref, mn_ref):
    a = a_ref[0]
    colid = lax.broadcasted_iota(jnp.int32, (N, N), 1)
    work = a
    vals = []
    idxs = []
    for _ in range(TOPK):
        m = jnp.max(work, axis=1, keepdims=True)
        eq = work == m
        idxk = jnp.min(jnp.where(eq, colid, N), axis=1, keepdims=True)
        vals.append(m)
        idxs.append(idxk)
        work = jnp.where(colid == idxk, -jnp.inf, work)
    v = jnp.concatenate(vals, axis=1)
    ix = jnp.concatenate(idxs, axis=1)
    inv = 1.0 / (jnp.sum(v, axis=1, keepdims=True) + 1e-12)
    an_ref[0] = v * inv
    idx_ref[0] = ix + pl.program_id(0) * N
    mn_ref[0] = jnp.where(work != a, a * inv, 0.0)


def _topk(A):
    return pl.pallas_call(
        _topk_body,
        grid=(NB,),
        in_specs=[pl.BlockSpec((1, N, N), lambda i: (i, 0, 0))],
        out_specs=[pl.BlockSpec((1, N, TOPK), lambda i: (i, 0, 0)),
                   pl.BlockSpec((1, N, TOPK), lambda i: (i, 0, 0)),
                   pl.BlockSpec((1, N, N), lambda i: (i, 0, 0))],
        out_shape=[jax.ShapeDtypeStruct((NB, N, TOPK), jnp.float32),
                   jax.ShapeDtypeStruct((NB, N, TOPK), jnp.int32),
                   jax.ShapeDtypeStruct((NB, N, N), jnp.float32)],
    )(A)


def _proj_body(x_ref, w_ref, b_ref, mn_ref, an_ref, cc_ref, pre_ref, lin_ref, xb_ref):
    x = x_ref[0]
    xl = lax.dot_general(
        x, w_ref[...], (((1,), (1,)), ((), ())),
        preferred_element_type=jnp.float32) + b_ref[...][None, :]
    gg = lax.dot_general(
        mn_ref[0], xl, (((1,), (0,)), ((), ())),
        preferred_element_type=jnp.float32)
    sume = jnp.sum(an_ref[0], axis=1, keepdims=True)
    c1 = cc_ref[0, :][None, :]
    c2 = cc_ref[1, :][None, :]
    xlc1 = xl * c1
    pre_ref[0] = xlc1 + c2
    xb_ref[0] = xlc1
    lin_ref[0] = 0.505 * ((sume * xl + gg) * c1 + sume * c2)


def _proj(X, W, b, Mn, anb, cc, din):
    return pl.pallas_call(
        _proj_body,
        grid=(NB,),
        in_specs=[pl.BlockSpec((1, N, din), lambda i: (i, 0, 0)),
                  pl.BlockSpec((MSG, din), lambda i: (0, 0)),
                  pl.BlockSpec((MSG,), lambda i: (0,)),
                  pl.BlockSpec((1, N, N), lambda i: (i, 0, 0)),
                  pl.BlockSpec((1, N, TOPK), lambda i: (i, 0, 0)),
                  pl.BlockSpec((2, MSG), lambda i: (0, 0))],
        out_specs=[pl.BlockSpec((1, N, MSG), lambda i: (i, 0, 0)),
                   pl.BlockSpec((1, N, MSG), lambda i: (i, 0, 0)),
                   pl.BlockSpec((1, N, MSG), lambda i: (i, 0, 0))],
        out_shape=[jax.ShapeDtypeStruct((NB, N, MSG), jnp.float32),
                   jax.ShapeDtypeStruct((NB, N, MSG), jnp.float32),
                   jax.ShapeDtypeStruct((NB, N, MSG), jnp.float32)],
    )(X, W, b, Mn, anb, cc)


def _update_body(x_ref, m_ref, l_ref, w_ref, g_ref, be_ref, o_ref, *, din):
    x = x_ref[0]
    m = l_ref[0] + 0.495 * m_ref[0]
    w = w_ref[...]
    u = lax.dot_general(x, w[:, :din], (((1,), (1,)), ((), ())),
                        preferred_element_type=jnp.float32)
    u = u + lax.dot_general(m, w[:, din:], (((1,), (1,)), ((), ())),
                            preferred_element_type=jnp.float32)
    u = u * (g_ref[...] * BN_SCALE)[None, :] + be_ref[...][None, :]
    u = jnp.where(u >= 0, u, 0.01 * u)
    nrm = jnp.maximum(jnp.sqrt(jnp.sum(u * u, axis=1, keepdims=True)), 1e-12)
    o_ref[0, :, :din] = x * INV_SQRT2
    o_ref[0, :, din:] = u * (INV_SQRT2 / nrm)


def _update(X, msg, lin, W, g, be, din):
    return pl.pallas_call(
        functools.partial(_update_body, din=din),
        grid=(NB,),
        in_specs=[pl.BlockSpec((1, N, din), lambda i: (i, 0, 0)),
                  pl.BlockSpec((1, N, MSG), lambda i: (i, 0, 0)),
                  pl.BlockSpec((1, N, MSG), lambda i: (i, 0, 0)),
                  pl.BlockSpec((MSG, din + MSG), lambda i: (0, 0)),
                  pl.BlockSpec((MSG,), lambda i: (0,)),
                  pl.BlockSpec((MSG,), lambda i: (0,))],
        out_specs=pl.BlockSpec((1, N, din + MSG), lambda i: (i, 0, 0)),
        out_shape=jax.ShapeDtypeStruct((NB, N, din + MSG), jnp.float32),
    )(X, msg, lin, W, g, be)


def _final_body(x_ref, m_ref, l_ref, w_ref, b_ref, o_ref, *, din):
    x = x_ref[0]
    m = l_ref[0] + 0.495 * m_ref[0]
    w = w_ref[...]
    u = lax.dot_general(x, w[:, :din], (((1,), (1,)), ((), ())),
                        preferred_element_type=jnp.float32)
    u = u + lax.dot_general(m, w[:, din:], (((1,), (1,)), ((), ())),
                            preferred_element_type=jnp.float32)
    o_ref[0] = u + b_ref[...][None, :]


def _final(X, msg, lin, W, b, din):
    return pl.pallas_call(
        functools.partial(_final_body, din=din),
        grid=(NB,),
        in_specs=[pl.BlockSpec((1, N, din), lambda i: (i, 0, 0)),
                  pl.BlockSpec((1, N, MSG), lambda i: (i, 0, 0)),
                  pl.BlockSpec((1, N, MSG), lambda i: (i, 0, 0)),
                  pl.BlockSpec((MSG, din + MSG), lambda i: (0, 0)),
                  pl.BlockSpec((MSG,), lambda i: (0,))],
        out_specs=pl.BlockSpec((1, N, MSG), lambda i: (i, 0, 0)),
        out_shape=jax.ShapeDtypeStruct((NB, N, MSG), jnp.float32),
    )(X, msg, lin, W, b)


def _sc_msg_body(xl_hbm, xb_hbm, idx_hbm, an_hbm, out_hbm,
                 idx_v, an_v, xi_v,
                 rows0, rows1, rows2, rows3, rows4, rows5, rows6, rows7,
                 msg0, msg1, msg2, msg3, msg4, msg5, msg6, msg7,
                 semg0, semg1, semg2, semg3, semg4, semg5, semg6, semg7,
                 semo0, semo1, semo2, semo3, semo4, semo5, semo6, semo7):
    c = lax.axis_index("c")
    s = lax.axis_index("s")
    wid = s * NC + c
    base = wid * NPW

    pltpu.sync_copy(idx_hbm.at[pl.ds(base, NPW)], idx_v)
    pltpu.sync_copy(an_hbm.at[pl.ds(base, NPW)], an_v)

    rows = [rows0, rows1, rows2, rows3, rows4, rows5, rows6, rows7]
    msgs = [msg0, msg1, msg2, msg3, msg4, msg5, msg6, msg7]
    semg = [semg0, semg1, semg2, semg3, semg4, semg5, semg6, semg7]
    semo = [semo0, semo1, semo2, semo3, semo4, semo5, semo6, semo7]
    for p in range(NBUF):
        pltpu.make_async_copy(xl_hbm.at[idx_v.at[p]], rows[p], semg[p]).start()

    def compute_node(n, rowsb, msgb):
        av = an_v[n, :]
        aks = [jnp.broadcast_to(av[k], (LANES,)) for k in range(TOPK)]
        nl = lax.rem(n, XCHUNK)

        def vloop(v, carry):
            sl = pl.ds(v * LANES, LANES)
            basev = xi_v[nl, sl]
            acc = jnp.zeros((LANES,), jnp.float32)
            for k in range(TOPK):
                h = jnp.abs(rowsb[k, sl] + basev)
                acc = acc + aks[k] * h
            msgb[0, sl] = acc
            return carry
        lax.fori_loop(0, VCHUNKS, vloop, 0, unroll=2)

    def do_node(n, rowsb, msgb, sg, so):
        pltpu.make_async_copy(xl_hbm.at[idx_v.at[n]], rowsb, sg).wait()

        @pl.when(n >= NBUF)
        def _drain():
            pltpu.make_async_copy(
                msgb, out_hbm.at[pl.ds(base + n - NBUF, 1)], so).wait()

        compute_node(n, rowsb, msgb)

        @pl.when(n + NBUF < NPW)
        def _prefetch():
            pltpu.make_async_copy(xl_hbm.at[idx_v.at[n + NBUF]], rowsb, sg).start()

        pltpu.make_async_copy(msgb, out_hbm.at[pl.ds(base + n, 1)], so).start()

    def group(t, carry):
        @pl.when(lax.rem(t, GPC) == 0)
        def _xi_refresh():
            cstart = lax.div(t, GPC) * XCHUNK
            pltpu.sync_copy(xb_hbm.at[pl.ds(base + cstart, XCHUNK)], xi_v)

        n0 = NBUF * t
        for p in range(NBUF):
            do_node(n0 + p, rows[p], msgs[p], semg[p], semo[p])
        return carry

    lax.fori_loop(0, NPW // NBUF, group, 0)
    for p in range(NBUF):
        pltpu.make_async_copy(
            msgs[p], out_hbm.at[pl.ds(base + NPW - NBUF + p, 1)], semo[p]).wait()


def _sc_msg(xl_flat, xb_flat, idx_flat, an_flat):
    return pl.kernel(
        _sc_msg_body,
        out_type=jax.ShapeDtypeStruct((TOTALH, MSG), jnp.float32),
        mesh=plsc.VectorSubcoreMesh(core_axis_name="c", subcore_axis_name="s",
                                    num_cores=NC, num_subcores=NS),
        scratch_types=[
            pltpu.VMEM((NPW, TOPK), jnp.int32),
            pltpu.VMEM((NPW, TOPK), jnp.float32),
            pltpu.VMEM((XCHUNK, MSG), jnp.float32),
            pltpu.VMEM((TOPK, MSG), jnp.float32),
            pltpu.VMEM((TOPK, MSG), jnp.float32),
            pltpu.VMEM((TOPK, MSG), jnp.float32),
            pltpu.VMEM((TOPK, MSG), jnp.float32),
            pltpu.VMEM((TOPK, MSG), jnp.float32),
            pltpu.VMEM((TOPK, MSG), jnp.float32),
            pltpu.VMEM((TOPK, MSG), jnp.float32),
            pltpu.VMEM((TOPK, MSG), jnp.float32),
            pltpu.VMEM((1, MSG), jnp.float32),
            pltpu.VMEM((1, MSG), jnp.float32),
            pltpu.VMEM((1, MSG), jnp.float32),
            pltpu.VMEM((1, MSG), jnp.float32),
            pltpu.VMEM((1, MSG), jnp.float32),
            pltpu.VMEM((1, MSG), jnp.float32),
            pltpu.VMEM((1, MSG), jnp.float32),
            pltpu.VMEM((1, MSG), jnp.float32),
        ] + [pltpu.SemaphoreType.DMA] * 16,
    )(xl_flat, xb_flat, idx_flat, an_flat)


def kernel(X_input, adjacency_matrix,
           W_agg0, b_agg0, g_agg0, be_agg0,
           W_agg1, b_agg1, g_agg1, be_agg1,
           W_agg2, b_agg2, g_agg2, be_agg2,
           W_upd0, g_upd0, be_upd0,
           W_upd1, g_upd1, be_upd1,
           W_fin, b_fin):
    A = adjacency_matrix.reshape(B, N, N)
    tk = [_topk(A[:NB]), _topk(A[NB:])]
    an_hh = [tk[0][0], tk[1][0]]
    Mn_h = [tk[0][2], tk[1][2]]
    an_h = [an_hh[0].reshape(TOTALH, TOPK), an_hh[1].reshape(TOTALH, TOPK)]
    idx_h = [tk[0][1].reshape(TOTALH, TOPK), tk[1][1].reshape(TOTALH, TOPK)]

    aggs = [(W_agg0, b_agg0, g_agg0, be_agg0),
            (W_agg1, b_agg1, g_agg1, be_agg1),
            (W_agg2, b_agg2, g_agg2, be_agg2)]
    upds = [(W_upd0, g_upd0, be_upd0), (W_upd1, g_upd1, be_upd1)]

    Xh = [X_input[:NB], X_input[NB:]]
    din = MSG
    outs = None
    for layer in range(3):
        W, b, g, be = aggs[layer]
        cc = jnp.stack([g * BN_SCALE, be])
        pl_ = [_proj(Xh[h], W, b, Mn_h[h], an_hh[h], cc, din)
               for h in range(2)]
        msg = [_sc_msg(pl_[h][0].reshape(TOTALH, MSG),
                       pl_[h][2].reshape(TOTALH, MSG),
                       idx_h[h], an_h[h]).reshape(NB, N, MSG)
               for h in range(2)]
        if layer < 2:
            Wu, gu, beu = upds[layer]
            Xh = [_update(Xh[h], msg[h], pl_[h][1], Wu, gu, beu, din)
                  for h in range(2)]
            din += MSG
        else:
            outs = [_final(Xh[h], msg[h], pl_[h][1], W_fin, b_fin, din)
                    for h in range(2)]
    return jnp.concatenate(outs, axis=0)


# speedup vs baseline: 1.2609x; 1.0971x over previous
import functools

import numpy as np
import jax
import jax.numpy as jnp
from jax import lax
from jax.experimental import pallas as pl
from jax.experimental.pallas import tpu as pltpu
from jax.experimental.pallas import tpu_sc as plsc

B = 8
NB = B // 2
N = 512
MSG = 512
TOPK = 16
EPS_BN = 1e-5
INV_SQRT2 = float(1.0 / np.sqrt(2.0))
BN_SCALE = float(1.0 / np.sqrt(1.0 + EPS_BN))

NC = 2
NS = 16
LANES = 16
NW = NC * NS
TOTALH = NB * N
NPW = TOTALH // NW
VCHUNKS = MSG // LANES
NBUF = 4
XCHUNK = 32
GPC = XCHUNK // NBUF


def _topk_body(a_ref, an_ref, idx_ref, mn_ref):
    a = a_ref[0]
    colid = lax.broadcasted_iota(jnp.int32, (N, N), 1)
    work = a
    vals = []
    idxs = []
    for _ in range(TOPK):
        m = jnp.max(work, axis=1, keepdims=True)
        eq = work == m
        idxk = jnp.min(jnp.where(eq, colid, N), axis=1, keepdims=True)
        vals.append(m)
        idxs.append(idxk)
        work = jnp.where(colid == idxk, -jnp.inf, work)
    v = jnp.concatenate(vals, axis=1)
    ix = jnp.concatenate(idxs, axis=1)
    inv = 1.0 / (jnp.sum(v, axis=1, keepdims=True) + 1e-12)
    an_ref[0] = v * inv
    idx_ref[0] = ix + pl.program_id(0) * N
    mn_ref[0] = jnp.where(work != a, a * inv, 0.0)


def _topk(A):
    return pl.pallas_call(
        _topk_body,
        grid=(NB,),
        in_specs=[pl.BlockSpec((1, N, N), lambda i: (i, 0, 0))],
        out_specs=[pl.BlockSpec((1, N, TOPK), lambda i: (i, 0, 0)),
                   pl.BlockSpec((1, N, TOPK), lambda i: (i, 0, 0)),
                   pl.BlockSpec((1, N, N), lambda i: (i, 0, 0))],
        out_shape=[jax.ShapeDtypeStruct((NB, N, TOPK), jnp.float32),
                   jax.ShapeDtypeStruct((NB, N, TOPK), jnp.int32),
                   jax.ShapeDtypeStruct((NB, N, N), jnp.float32)],
    )(A)


def _proj_body(x_ref, w_ref, b_ref, mn_ref, an_ref, cc_ref, pre_ref, lin_ref, xb_ref):
    x = x_ref[0]
    xl = lax.dot_general(
        x, w_ref[...], (((1,), (1,)), ((), ())),
        preferred_element_type=jnp.float32) + b_ref[...][None, :]
    gg = lax.dot_general(
        mn_ref[0], xl, (((1,), (0,)), ((), ())),
        preferred_element_type=jnp.float32)
    sume = jnp.sum(an_ref[0], axis=1, keepdims=True)
    c1 = cc_ref[0, :][None, :]
    c2 = cc_ref[1, :][None, :]
    xlc1 = xl * c1
    pre_ref[0] = xlc1 + c2
    xb_ref[0] = xlc1
    lin_ref[0] = 0.505 * ((sume * xl + gg) * c1 + sume * c2)


def _proj(X, W, b, Mn, anb, cc, din):
    return pl.pallas_call(
        _proj_body,
        grid=(NB,),
        in_specs=[pl.BlockSpec((1, N, din), lambda i: (i, 0, 0)),
                  pl.BlockSpec((MSG, din), lambda i: (0, 0)),
                  pl.BlockSpec((MSG,), lambda i: (0,)),
                  pl.BlockSpec((1, N, N), lambda i: (i, 0, 0)),
                  pl.BlockSpec((1, N, TOPK), lambda i: (i, 0, 0)),
                  pl.BlockSpec((2, MSG), lambda i: (0, 0))],
        out_specs=[pl.BlockSpec((1, N, MSG), lambda i: (i, 0, 0)),
                   pl.BlockSpec((1, N, MSG), lambda i: (i, 0, 0)),
                   pl.BlockSpec((1, N, MSG), lambda i: (i, 0, 0))],
        out_shape=[jax.ShapeDtypeStruct((NB, N, MSG), jnp.float32),
                   jax.ShapeDtypeStruct((NB, N, MSG), jnp.float32),
                   jax.ShapeDtypeStruct((NB, N, MSG), jnp.float32)],
    )(X, W, b, Mn, anb, cc)


def _update_body(x_ref, m_ref, l_ref, w_ref, g_ref, be_ref, o_ref, *, din):
    x = x_ref[0]
    m = l_ref[0] + 0.495 * m_ref[0]
    w = w_ref[...]
    u = lax.dot_general(x, w[:, :din], (((1,), (1,)), ((), ())),
                        preferred_element_type=jnp.float32)
    u = u + lax.dot_general(m, w[:, din:], (((1,), (1,)), ((), ())),
                            preferred_element_type=jnp.float32)
    u = u * (g_ref[...] * BN_SCALE)[None, :] + be_ref[...][None, :]
    u = jnp.where(u >= 0, u, 0.01 * u)
    nrm = jnp.maximum(jnp.sqrt(jnp.sum(u * u, axis=1, keepdims=True)), 1e-12)
    o_ref[0, :, :din] = x * INV_SQRT2
    o_ref[0, :, din:] = u * (INV_SQRT2 / nrm)


def _update(X, msg, lin, W, g, be, din):
    return pl.pallas_call(
        functools.partial(_update_body, din=din),
        grid=(NB,),
        in_specs=[pl.BlockSpec((1, N, din), lambda i: (i, 0, 0)),
                  pl.BlockSpec((1, N, MSG), lambda i: (i, 0, 0)),
                  pl.BlockSpec((1, N, MSG), lambda i: (i, 0, 0)),
                  pl.BlockSpec((MSG, din + MSG), lambda i: (0, 0)),
                  pl.BlockSpec((MSG,), lambda i: (0,)),
                  pl.BlockSpec((MSG,), lambda i: (0,))],
        out_specs=pl.BlockSpec((1, N, din + MSG), lambda i: (i, 0, 0)),
        out_shape=jax.ShapeDtypeStruct((NB, N, din + MSG), jnp.float32),
    )(X, msg, lin, W, g, be)


def _final_body(x_ref, m_ref, l_ref, w_ref, b_ref, o_ref, *, din):
    x = x_ref[0]
    m = l_ref[0] + 0.495 * m_ref[0]
    w = w_ref[...]
    u = lax.dot_general(x, w[:, :din], (((1,), (1,)), ((), ())),
                        preferred_element_type=jnp.float32)
    u = u + lax.dot_general(m, w[:, din:], (((1,), (1,)), ((), ())),
                            preferred_element_type=jnp.float32)
    o_ref[0] = u + b_ref[...][None, :]


def _final(X, msg, lin, W, b, din):
    return pl.pallas_call(
        functools.partial(_final_body, din=din),
        grid=(NB,),
        in_specs=[pl.BlockSpec((1, N, din), lambda i: (i, 0, 0)),
                  pl.BlockSpec((1, N, MSG), lambda i: (i, 0, 0)),
                  pl.BlockSpec((1, N, MSG), lambda i: (i, 0, 0)),
                  pl.BlockSpec((MSG, din + MSG), lambda i: (0, 0)),
                  pl.BlockSpec((MSG,), lambda i: (0,))],
        out_specs=pl.BlockSpec((1, N, MSG), lambda i: (i, 0, 0)),
        out_shape=jax.ShapeDtypeStruct((NB, N, MSG), jnp.float32),
    )(X, msg, lin, W, b)


def _sc_msg_body(xl_hbm, xb_hbm, idx_hbm, an_hbm, out_hbm,
                 idx_v, an_v, xi_v, rows0, rows1, rows2, rows3,
                 msg0, msg1, msg2, msg3,
                 semg0, semg1, semg2, semg3, semo0, semo1, semo2, semo3):
    c = lax.axis_index("c")
    s = lax.axis_index("s")
    wid = s * NC + c
    base = wid * NPW

    pltpu.sync_copy(idx_hbm.at[pl.ds(base, NPW)], idx_v)
    pltpu.sync_copy(an_hbm.at[pl.ds(base, NPW)], an_v)

    rows = [rows0, rows1, rows2, rows3]
    msgs = [msg0, msg1, msg2, msg3]
    semg = [semg0, semg1, semg2, semg3]
    semo = [semo0, semo1, semo2, semo3]
    for p in range(NBUF):
        pltpu.make_async_copy(xl_hbm.at[idx_v.at[p]], rows[p], semg[p]).start()

    def compute_node(n, rowsb, msgb):
        av = an_v[n, :]
        aks = [jnp.broadcast_to(av[k], (LANES,)) for k in range(TOPK)]
        nl = lax.rem(n, XCHUNK)

        def vloop(v, carry):
            sl = pl.ds(v * LANES, LANES)
            basev = xi_v[nl, sl]
            acc = jnp.zeros((LANES,), jnp.float32)
            for k in range(TOPK):
                h = jnp.abs(rowsb[k, sl] + basev)
                acc = acc + aks[k] * h
            msgb[0, sl] = acc
            return carry
        lax.fori_loop(0, VCHUNKS, vloop, 0, unroll=2)

    def do_node(n, rowsb, msgb, sg, so):
        pltpu.make_async_copy(xl_hbm.at[idx_v.at[n]], rowsb, sg).wait()

        @pl.when(n >= NBUF)
        def _drain():
            pltpu.make_async_copy(
                msgb, out_hbm.at[pl.ds(base + n - NBUF, 1)], so).wait()

        compute_node(n, rowsb, msgb)

        @pl.when(n + NBUF < NPW)
        def _prefetch():
            pltpu.make_async_copy(xl_hbm.at[idx_v.at[n + NBUF]], rowsb, sg).start()

        pltpu.make_async_copy(msgb, out_hbm.at[pl.ds(base + n, 1)], so).start()

    def group(t, carry):
        @pl.when(lax.rem(t, GPC) == 0)
        def _xi_refresh():
            cstart = lax.div(t, GPC) * XCHUNK
            pltpu.sync_copy(xb_hbm.at[pl.ds(base + cstart, XCHUNK)], xi_v)

        n0 = NBUF * t
        for p in range(NBUF):
            do_node(n0 + p, rows[p], msgs[p], semg[p], semo[p])
        return carry

    lax.fori_loop(0, NPW // NBUF, group, 0)
    for p in range(NBUF):
        pltpu.make_async_copy(
            msgs[p], out_hbm.at[pl.ds(base + NPW - NBUF + p, 1)], semo[p]).wait()


def _sc_msg(xl_flat, xb_flat, idx_flat, an_flat):
    return pl.kernel(
        _sc_msg_body,
        out_type=jax.ShapeDtypeStruct((TOTALH, MSG), jnp.float32),
        mesh=plsc.VectorSubcoreMesh(core_axis_name="c", subcore_axis_name="s",
                                    num_cores=NC, num_subcores=NS),
        scratch_types=[
            pltpu.VMEM((NPW, TOPK), jnp.int32),
            pltpu.VMEM((NPW, TOPK), jnp.float32),
            pltpu.VMEM((XCHUNK, MSG), jnp.float32),
            pltpu.VMEM((TOPK, MSG), jnp.float32),
            pltpu.VMEM((TOPK, MSG), jnp.float32),
            pltpu.VMEM((TOPK, MSG), jnp.float32),
            pltpu.VMEM((TOPK, MSG), jnp.float32),
            pltpu.VMEM((1, MSG), jnp.float32),
            pltpu.VMEM((1, MSG), jnp.float32),
            pltpu.VMEM((1, MSG), jnp.float32),
            pltpu.VMEM((1, MSG), jnp.float32),
        ] + [pltpu.SemaphoreType.DMA] * 8,
    )(xl_flat, xb_flat, idx_flat, an_flat)


def kernel(X_input, adjacency_matrix,
           W_agg0, b_agg0, g_agg0, be_agg0,
           W_agg1, b_agg1, g_agg1, be_agg1,
           W_agg2, b_agg2, g_agg2, be_agg2,
           W_upd0, g_upd0, be_upd0,
           W_upd1, g_upd1, be_upd1,
           W_fin, b_fin):
    A = adjacency_matrix.reshape(B, N, N)
    tk = [_topk(A[:NB]), _topk(A[NB:])]
    an_hh = [tk[0][0], tk[1][0]]
    Mn_h = [tk[0][2], tk[1][2]]
    an_h = [an_hh[0].reshape(TOTALH, TOPK), an_hh[1].reshape(TOTALH, TOPK)]
    idx_h = [tk[0][1].reshape(TOTALH, TOPK), tk[1][1].reshape(TOTALH, TOPK)]

    aggs = [(W_agg0, b_agg0, g_agg0, be_agg0),
            (W_agg1, b_agg1, g_agg1, be_agg1),
            (W_agg2, b_agg2, g_agg2, be_agg2)]
    upds = [(W_upd0, g_upd0, be_upd0), (W_upd1, g_upd1, be_upd1)]

    Xh = [X_input[:NB], X_input[NB:]]
    din = MSG
    outs = None
    for layer in range(3):
        W, b, g, be = aggs[layer]
        cc = jnp.stack([g * BN_SCALE, be])
        pl_ = [_proj(Xh[h], W, b, Mn_h[h], an_hh[h], cc, din)
               for h in range(2)]
        msg = [_sc_msg(pl_[h][0].reshape(TOTALH, MSG),
                       pl_[h][2].reshape(TOTALH, MSG),
                       idx_h[h], an_h[h]).reshape(NB, N, MSG)
               for h in range(2)]
        if layer < 2:
            Wu, gu, beu = upds[layer]
            Xh = [_update(Xh[h], msg[h], pl_[h][1], Wu, gu, beu, din)
                  for h in range(2)]
            din += MSG
        else:
            outs = [_final(Xh[h], msg[h], pl_[h][1], W_fin, b_fin, din)
                    for h in range(2)]
    return jnp.concatenate(outs, axis=0)
